# baseline (speedup 1.0000x reference)
import functools

import jax
import jax.numpy as jnp
from jax.experimental import pallas as pl
from jax.experimental.pallas import tpu as pltpu

_G = 8


def _fused_kernel(xr_ref, wbig_ref, pbig_ref, nsum_ref, ssum_ref, bcast_ref,
                  out_ref):
    t = jnp.dot(xr_ref[...], wbig_ref[...], preferred_element_type=jnp.float32)
    u = jnp.tanh(t)
    s = jnp.dot(u * u, nsum_ref[...], preferred_element_type=jnp.float32)
    r = jax.lax.rsqrt(jnp.maximum(s, 1e-12))
    d = jnp.dot(u, pbig_ref[...], preferred_element_type=jnp.float32)
    rb = jnp.dot(r, bcast_ref[...], preferred_element_type=jnp.float32)
    e = jnp.exp(d * rb)
    denom = jnp.dot(e, ssum_ref[...], preferred_element_type=jnp.float32)
    ob = jnp.dot(1.0 / denom, bcast_ref[...],
                 preferred_element_type=jnp.float32)
    out_ref[...] = e * ob


@functools.partial(jax.jit, static_argnames=("block_m",))
def _run(xr, wbig, pbig, nsum, ssum, bcast, block_m):
    n_packed, dk = xr.shape
    bm = block_m // _G
    grid = (n_packed // bm,)
    return pl.pallas_call(
        _fused_kernel,
        grid=grid,
        in_specs=[
            pl.BlockSpec((bm, dk), lambda i: (i, 0)),
            pl.BlockSpec(wbig.shape, lambda i: (0, 0)),
            pl.BlockSpec(pbig.shape, lambda i: (0, 0)),
            pl.BlockSpec(nsum.shape, lambda i: (0, 0)),
            pl.BlockSpec(ssum.shape, lambda i: (0, 0)),
            pl.BlockSpec(bcast.shape, lambda i: (0, 0)),
        ],
        out_specs=pl.BlockSpec((bm, _G * 8), lambda i: (i, 0)),
        out_shape=jax.ShapeDtypeStruct((n_packed, _G * 8), jnp.float32),
        compiler_params=pltpu.CompilerParams(
            dimension_semantics=("parallel",),
        ),
    )(xr, wbig, pbig, nsum, ssum, bcast)


def kernel(x, W, prototypes, hamming_scale):
    b, s, d = x.shape
    k = prototypes.shape[0]
    n_rows = b * s
    xr = x.reshape(n_rows // _G, _G * d)
    eye = jnp.eye(_G, dtype=jnp.float32)
    wbig = jnp.kron(eye, W.T)
    pn = prototypes / jnp.maximum(
        jnp.linalg.norm(prototypes, axis=-1, keepdims=True), 1e-12
    )
    pnt = (3.0 * jnp.asarray(hamming_scale, jnp.float32)) * pn.T
    pbig = jnp.kron(eye, pnt)
    nsum = jnp.kron(eye, jnp.ones((6, 1), jnp.float32))
    ssum = jnp.kron(eye, jnp.ones((k, 1), jnp.float32))
    bcast = jnp.kron(eye, jnp.ones((1, k), jnp.float32))
    out = _run(xr, wbig, pbig, nsum, ssum, bcast, block_m=2048)
    return out.reshape(b, s, k)


# speedup vs baseline: 4.4482x; 4.4482x over previous
import functools

import jax
import jax.numpy as jnp
from jax.experimental import pallas as pl
from jax.experimental.pallas import tpu as pltpu


def _fused_kernel(x_ref, wt_ref, pns_ref, out_ref):
    t = jnp.dot(x_ref[...], wt_ref[...], preferred_element_type=jnp.float32)
    tt = t.T
    u = jnp.tanh(tt)
    s = jnp.sum(u * u, axis=0, keepdims=True)
    r = jax.lax.rsqrt(jnp.maximum(s, 1e-12))
    d = jnp.dot(pns_ref[...], u, preferred_element_type=jnp.float32)
    e = jnp.exp(d * r)
    out_ref[...] = e / jnp.sum(e, axis=0, keepdims=True)


@functools.partial(jax.jit, static_argnames=("block_m",))
def _run(x2d, wt, pns, block_m):
    n_rows, dk = x2d.shape
    grid = (n_rows // block_m,)
    return pl.pallas_call(
        _fused_kernel,
        grid=grid,
        in_specs=[
            pl.BlockSpec((block_m, dk), lambda i: (i, 0)),
            pl.BlockSpec(wt.shape, lambda i: (0, 0)),
            pl.BlockSpec(pns.shape, lambda i: (0, 0)),
        ],
        out_specs=pl.BlockSpec((8, block_m), lambda i: (0, i)),
        out_shape=jax.ShapeDtypeStruct((8, n_rows), jnp.float32),
        compiler_params=pltpu.CompilerParams(
            dimension_semantics=("parallel",),
        ),
    )(x2d, wt, pns)


def kernel(x, W, prototypes, hamming_scale):
    b, s, d = x.shape
    k = prototypes.shape[0]
    x2d = x.reshape(b * s, d)
    pn = prototypes / jnp.maximum(
        jnp.linalg.norm(prototypes, axis=-1, keepdims=True), 1e-12
    )
    pns = (3.0 * jnp.asarray(hamming_scale, jnp.float32)) * pn
    out = _run(x2d, W.T, pns, block_m=2048)
    return out.T.reshape(b, s, k)
